# baseline (speedup 1.0000x reference)
import jax
import jax.numpy as jnp
from jax.experimental import pallas as pl
from jax.experimental.pallas import tpu as pltpu

_NUM_GLOBAL = 576
_NUM_LOCAL = 196
_NB = 4


def _manual_kernel(B, P, D):
    S = B // _NB

    def body(x_hbm, cls_ref, pos_ref, out_hbm, xbuf, obuf, in_sems, out_sems):
        s = pl.program_id(0)

        def in_copy(step, slot):
            return pltpu.make_async_copy(
                x_hbm.at[pl.ds(step * _NB, _NB)], xbuf.at[slot],
                in_sems.at[slot])

        def out_copy(step, slot):
            return pltpu.make_async_copy(
                obuf.at[slot], out_hbm.at[pl.ds(step * _NB, _NB)],
                out_sems.at[slot])

        @pl.when(s == 0)
        def _():
            in_copy(0, 0).start()

        @pl.when(s < S - 1)
        def _():
            in_copy(s + 1, (s + 1) % 2).start()

        in_copy(s, s % 2).wait()

        @pl.when(s >= 2)
        def _():
            out_copy(s - 2, s % 2).wait()

        for i in range(_NB):
            left = jnp.concatenate([cls_ref[...], xbuf[s % 2, i]], axis=0)
            obuf[s % 2, i] = jnp.concatenate([left, pos_ref[...]], axis=1)

        out_copy(s, s % 2).start()

        @pl.when(s == S - 1)
        def _():
            out_copy(s - 1, (s - 1) % 2).wait()
            out_copy(s, s % 2).wait()

    return pl.pallas_call(
        body,
        grid=(S,),
        in_specs=[
            pl.BlockSpec(memory_space=pl.ANY),
            pl.BlockSpec((1, D), lambda s: (0, 0)),
            pl.BlockSpec((P + 1, D), lambda s: (0, 0)),
        ],
        out_specs=pl.BlockSpec(memory_space=pl.ANY),
        out_shape=jax.ShapeDtypeStruct((B, P + 1, 2 * D), jnp.float32),
        scratch_shapes=[
            pltpu.VMEM((2, _NB, P, D), jnp.float32),
            pltpu.VMEM((2, _NB, P + 1, 2 * D), jnp.float32),
            pltpu.SemaphoreType.DMA((2,)),
            pltpu.SemaphoreType.DMA((2,)),
        ],
    )


def _fallback_body(x_ref, cls_ref, pos_ref, out_ref):
    left = jnp.concatenate([cls_ref[...], x_ref[0]], axis=0)
    out_ref[0] = jnp.concatenate([left, pos_ref[...]], axis=1)


def _fallback_kernel(B, P, D, E, dtype):
    return pl.pallas_call(
        _fallback_body,
        grid=(B,),
        in_specs=[
            pl.BlockSpec((1, P, D), lambda b: (b, 0, 0)),
            pl.BlockSpec((1, D), lambda b: (0, 0)),
            pl.BlockSpec((P + 1, E), lambda b: (0, 0)),
        ],
        out_specs=pl.BlockSpec((1, P + 1, D + E), lambda b: (b, 0, 0)),
        out_shape=jax.ShapeDtypeStruct((B, P + 1, D + E), dtype),
    )


def kernel(x, cls_embedding, pos_embedding_global, pos_embedding_local):
    B, P, D = x.shape
    if P == _NUM_GLOBAL:
        pos = pos_embedding_global
    elif P == _NUM_LOCAL:
        pos = pos_embedding_local
    else:
        raise RuntimeError(f"Num patches {P} not matching")
    E = pos.shape[1]

    if D == E and B % _NB == 0 and B // _NB >= 3 and x.dtype == jnp.float32:
        return _manual_kernel(B, P, D)(x, cls_embedding, pos)
    return _fallback_kernel(B, P, D, E, x.dtype)(x, cls_embedding, pos)


# speedup vs baseline: 1.0017x; 1.0017x over previous
import jax
import jax.numpy as jnp
from jax.experimental import pallas as pl

_NUM_GLOBAL = 576
_NUM_LOCAL = 196
_NBATCH = 4


def _body(x_ref, cls_ref, pos_ref, out_ref):
    for i in range(_NBATCH):
        left = jnp.concatenate([cls_ref[...], x_ref[i]], axis=0)
        out_ref[i] = jnp.concatenate([left, pos_ref[...]], axis=1)


def kernel(x, cls_embedding, pos_embedding_global, pos_embedding_local):
    B, P, D = x.shape
    if P == _NUM_GLOBAL:
        pos = pos_embedding_global
    elif P == _NUM_LOCAL:
        pos = pos_embedding_local
    else:
        raise RuntimeError(f"Num patches {P} not matching")
    E = pos.shape[1]
    nb = _NBATCH if B % _NBATCH == 0 else 1

    out = pl.pallas_call(
        _body if nb == _NBATCH else _body1,
        grid=(B // nb,),
        in_specs=[
            pl.BlockSpec((nb, P, D), lambda b: (b, 0, 0)),
            pl.BlockSpec((1, D), lambda b: (0, 0)),
            pl.BlockSpec((P + 1, E), lambda b: (0, 0)),
        ],
        out_specs=pl.BlockSpec((nb, P + 1, D + E), lambda b: (b, 0, 0)),
        out_shape=jax.ShapeDtypeStruct((B, P + 1, D + E), x.dtype),
    )(x, cls_embedding, pos)
    return out


def _body1(x_ref, cls_ref, pos_ref, out_ref):
    left = jnp.concatenate([cls_ref[...], x_ref[0]], axis=0)
    out_ref[0] = jnp.concatenate([left, pos_ref[...]], axis=1)
